# Initial kernel scaffold; baseline (speedup 1.0000x reference)
import jax
import jax.numpy as jnp
from jax.experimental import pallas as pl


def kernel(sentence, query_sentence, query_idf, emb, W1, b1, W2, b2, Wg, Wo, bo):
    raise NotImplementedError("write your pallas kernel here")


# speedup vs baseline: 3.0561x; 3.0561x over previous
import functools

import numpy as np

import jax
import jax.numpy as jnp
from jax import lax
from jax.experimental import pallas as pl
from jax.experimental.pallas import tpu as pltpu
from jax.experimental.pallas import tpu_sc as plsc

B, TQ, TD = 128, 20, 2048
VOCAB, EMB = 100000, 128
NBINS, NODES = 30, 5

_UBS = [float(x) for x in np.linspace(-1.0, 1.0, NBINS + 1)[1:]]


_SC_CORES, _SC_SUBCORES = 2, 16
_NW = _SC_CORES * _SC_SUBCORES
_CHUNK = 128
_D_ROWS_W = (B * TD) // _NW
_D_CHUNKS_W = _D_ROWS_W // _CHUNK
_Q_ROWS_W = (B * TQ) // _NW


@functools.cache
def _sc_gather():
    @functools.partial(
        pl.kernel,
        out_type=[
            jax.ShapeDtypeStruct((B * TD, EMB), jnp.float32),
            jax.ShapeDtypeStruct((B * TQ, EMB), jnp.float32),
        ],
        mesh=plsc.VectorSubcoreMesh(
            core_axis_name="c", subcore_axis_name="s", num_cores=_SC_CORES),
        scratch_types=[
            pltpu.VMEM((_D_CHUNKS_W, _CHUNK), jnp.int32),
            pltpu.VMEM((_CHUNK, EMB), jnp.float32),
            pltpu.VMEM((_Q_ROWS_W,), jnp.int32),
            pltpu.VMEM((_Q_ROWS_W, EMB), jnp.float32),
            pltpu.SemaphoreType.DMA,
        ],
    )
    def gather(table_hbm, didx_hbm, qidx_hbm, dout_hbm, qout_hbm,
               didx_v, rows_v, qidx_v, qrows_v, sem):
        wid = lax.axis_index("s") * _SC_CORES + lax.axis_index("c")

        pltpu.sync_copy(
            didx_hbm.at[pl.ds(wid * _D_CHUNKS_W, _D_CHUNKS_W)], didx_v)

        def chunk_body(i, _):
            pltpu.async_copy(table_hbm.at[didx_v.at[i]], rows_v, sem).wait()
            pltpu.sync_copy(
                rows_v,
                dout_hbm.at[pl.ds(wid * _D_ROWS_W + i * _CHUNK, _CHUNK)])
            return 0

        lax.fori_loop(0, _D_CHUNKS_W, chunk_body, 0)

        pltpu.sync_copy(qidx_hbm.at[wid], qidx_v)
        pltpu.async_copy(table_hbm.at[qidx_v], qrows_v, sem).wait()
        pltpu.sync_copy(qrows_v, qout_hbm.at[pl.ds(wid * _Q_ROWS_W, _Q_ROWS_W)])

    return gather


def _tc_body(d_emb_ref, q_emb_ref, d_tok_ref, q_tok_ref, q_idf_ref,
             w1_ref, b1_ref, w2_ref, consts_ref, out_ref):
    de = d_emb_ref[0]
    qe = q_emb_ref[0]

    dn = de / jnp.maximum(
        jnp.sqrt(jnp.sum(de * de, axis=1, keepdims=True)), 1e-8)
    qn = qe / jnp.maximum(
        jnp.sqrt(jnp.sum(qe * qe, axis=1, keepdims=True)), 1e-8)

    sim = lax.dot_general(qn, dn, (((1,), (1,)), ((), ())),
                          preferred_element_type=jnp.float32)

    dmask0 = (d_tok_ref[0] == 0).astype(jnp.float32)
    sim = sim + dmask0 * 1e7

    cum = [jnp.sum((sim < ub).astype(jnp.float32), axis=1, keepdims=True)
           for ub in _UBS]
    exact = jnp.sum(((sim > 0.999) & (sim < 1.001)).astype(jnp.float32),
                    axis=1, keepdims=True)
    cols = [cum[0]] + [cum[j] - cum[j - 1] for j in range(1, NBINS)] + [exact]
    hist = jnp.log(jnp.concatenate(cols, axis=1) + 1.0)

    h = jnp.tanh(
        lax.dot_general(hist, w1_ref[...], (((1,), (1,)), ((), ())),
                        preferred_element_type=jnp.float32) + b1_ref[...])
    b2s = consts_ref[0]
    wgs = consts_ref[1]
    wos = consts_ref[2]
    bos = consts_ref[3]

    ffw = jnp.tanh(
        jnp.sum(h * w2_ref[...], axis=1, keepdims=True) + b2s)

    qmask0 = (q_tok_ref[0] == 0).astype(jnp.float32)
    logits = q_idf_ref[0] * wgs + qmask0 * (-1e7)
    m = jnp.max(logits, axis=(0, 1), keepdims=True)
    e = jnp.exp(logits - m)
    w = e / jnp.sum(e, axis=(0, 1), keepdims=True)
    x = jnp.sum(w * ffw, axis=(0, 1), keepdims=True)
    out_ref[...] = (x * wos + bos).reshape(1, 1, 1)


_TC_IN_SPECS = [
    pl.BlockSpec((1, TD, EMB), lambda b: (b, 0, 0)),
    pl.BlockSpec((1, TQ, EMB), lambda b: (b, 0, 0)),
    pl.BlockSpec((1, 1, TD), lambda b: (b, 0, 0)),
    pl.BlockSpec((1, TQ, 1), lambda b: (b, 0, 0)),
    pl.BlockSpec((1, TQ, 1), lambda b: (b, 0, 0)),
    pl.BlockSpec((NODES, NBINS + 1), lambda b: (0, 0)),
    pl.BlockSpec((1, NODES), lambda b: (0, 0)),
    pl.BlockSpec((1, NODES), lambda b: (0, 0)),
    pl.BlockSpec(memory_space=pltpu.SMEM),
]
_TC_OUT_SPEC = pl.BlockSpec((1, 1, 1), lambda b: (b, 0, 0))


def kernel(sentence, query_sentence, query_idf, emb, W1, b1, W2, b2, Wg, Wo, bo):
    sentence = sentence.astype(jnp.int32)
    query_sentence = query_sentence.astype(jnp.int32)

    didx = sentence.reshape(-1).reshape((B * TD) // _CHUNK, _CHUNK)
    qidx = query_sentence.reshape(-1).reshape(_NW, _Q_ROWS_W)

    d_rows, q_rows = _sc_gather()(emb, didx, qidx)
    d_emb = d_rows.reshape(B, TD, EMB)
    q_emb = q_rows.reshape(B, TQ, EMB)

    score = pl.pallas_call(
        _tc_body,
        grid=(B,),
        in_specs=_TC_IN_SPECS,
        out_specs=_TC_OUT_SPEC,
        out_shape=jax.ShapeDtypeStruct((B, 1, 1), jnp.float32),
    )(d_emb, q_emb,
      sentence.reshape(B, 1, TD),
      query_sentence.reshape(B, TQ, 1),
      query_idf.reshape(B, TQ, 1),
      W1, b1.reshape(1, NODES), W2,
      jnp.stack([b2[0], Wg[0, 0], Wo[0, 0], bo[0]]))
    return score.reshape(B, 1)


# speedup vs baseline: 4.2354x; 1.3859x over previous
import functools

import numpy as np

import jax
import jax.numpy as jnp
from jax import lax
from jax.experimental import pallas as pl
from jax.experimental.pallas import tpu as pltpu
from jax.experimental.pallas import tpu_sc as plsc

B, TQ, TD = 128, 20, 2048
VOCAB, EMB = 100000, 128
NBINS, NODES = 30, 5

_UBS = [float(x) for x in np.linspace(-1.0, 1.0, NBINS + 1)[1:]]


_SC_CORES, _SC_SUBCORES = 2, 16
_NW = _SC_CORES * _SC_SUBCORES
_CHUNK = 128

_CB = 32
_NCHUNK = B // _CB
_DC_ROWS_W = (_CB * TD) // _NW
_DC_CHUNKS_W = _DC_ROWS_W // _CHUNK
_Q_ROWS_W = (B * TQ) // _NW


@functools.cache
def _sc_gather_d():
    @functools.partial(
        pl.kernel,
        out_type=jax.ShapeDtypeStruct((_CB * TD, EMB), jnp.float32),
        mesh=plsc.VectorSubcoreMesh(
            core_axis_name="c", subcore_axis_name="s", num_cores=_SC_CORES),
        scratch_types=[
            pltpu.VMEM((_DC_CHUNKS_W, _CHUNK), jnp.int32),
            pltpu.VMEM((_CHUNK, EMB), jnp.float32),
            pltpu.SemaphoreType.DMA,
        ],
    )
    def gather(table_hbm, didx_hbm, dout_hbm, didx_v, rows_v, sem):
        wid = lax.axis_index("s") * _SC_CORES + lax.axis_index("c")

        pltpu.sync_copy(
            didx_hbm.at[pl.ds(wid * _DC_CHUNKS_W, _DC_CHUNKS_W)], didx_v)

        def chunk_body(i, _):
            pltpu.async_copy(table_hbm.at[didx_v.at[i]], rows_v, sem).wait()
            pltpu.sync_copy(
                rows_v,
                dout_hbm.at[pl.ds((wid * _DC_CHUNKS_W + i) * _CHUNK, _CHUNK)])
            return 0

        lax.fori_loop(0, _DC_CHUNKS_W, chunk_body, 0)

    return gather


@functools.cache
def _sc_gather_q():
    @functools.partial(
        pl.kernel,
        out_type=jax.ShapeDtypeStruct((B * TQ, EMB), jnp.float32),
        mesh=plsc.VectorSubcoreMesh(
            core_axis_name="c", subcore_axis_name="s", num_cores=_SC_CORES),
        scratch_types=[
            pltpu.VMEM((_Q_ROWS_W,), jnp.int32),
            pltpu.VMEM((_Q_ROWS_W, EMB), jnp.float32),
            pltpu.SemaphoreType.DMA,
        ],
    )
    def gather(table_hbm, qidx_hbm, qout_hbm, qidx_v, qrows_v, sem):
        wid = lax.axis_index("s") * _SC_CORES + lax.axis_index("c")
        pltpu.sync_copy(qidx_hbm.at[wid], qidx_v)
        pltpu.async_copy(table_hbm.at[qidx_v], qrows_v, sem).wait()
        pltpu.sync_copy(qrows_v, qout_hbm.at[pl.ds(wid * _Q_ROWS_W, _Q_ROWS_W)])

    return gather


def _tc_body(d_emb_ref, q_emb_ref, d_tok_ref, q_tok_ref, q_idf_ref,
             w1_ref, b1_ref, w2_ref, consts_ref, out_ref):
    de = d_emb_ref[0]
    qe = q_emb_ref[0]

    qn = qe / jnp.maximum(
        jnp.sqrt(jnp.sum(qe * qe, axis=1, keepdims=True)), 1e-8)

    ones_row = jnp.ones((1, EMB), jnp.float32)
    dnorm2 = lax.dot_general(ones_row, de * de, (((1,), (1,)), ((), ())),
                             preferred_element_type=jnp.float32)
    rd = 1.0 / jnp.maximum(jnp.sqrt(dnorm2), 1e-8)

    sim = lax.dot_general(qn, de, (((1,), (1,)), ((), ())),
                          preferred_element_type=jnp.float32)
    sim = sim * rd

    dmask0 = (d_tok_ref[0] == 0).astype(jnp.float32)
    sim = sim + dmask0 * 1e7

    cum = [jnp.sum((sim < ub).astype(jnp.float32), axis=1, keepdims=True)
           for ub in _UBS]
    exact = jnp.sum(((sim > 0.999) & (sim < 1.001)).astype(jnp.float32),
                    axis=1, keepdims=True)
    cols = [cum[0]] + [cum[j] - cum[j - 1] for j in range(1, NBINS)] + [exact]
    hist = jnp.log(jnp.concatenate(cols, axis=1) + 1.0)

    h = jnp.tanh(
        lax.dot_general(hist, w1_ref[...], (((1,), (1,)), ((), ())),
                        preferred_element_type=jnp.float32) + b1_ref[...])
    b2s = consts_ref[0]
    wgs = consts_ref[1]
    wos = consts_ref[2]
    bos = consts_ref[3]

    ffw = jnp.tanh(
        jnp.sum(h * w2_ref[...], axis=1, keepdims=True) + b2s)

    qmask0 = (q_tok_ref[0] == 0).astype(jnp.float32)
    logits = q_idf_ref[0] * wgs + qmask0 * (-1e7)
    m = jnp.max(logits, axis=(0, 1), keepdims=True)
    e = jnp.exp(logits - m)
    w = e / jnp.sum(e, axis=(0, 1), keepdims=True)
    x = jnp.sum(w * ffw, axis=(0, 1), keepdims=True)
    out_ref[...] = (x * wos + bos).reshape(1, 1, 1)


def _tc_chunk(c, d_emb_c, q_emb, sent3, qtok3, qidf3, W1, b1r, W2, consts):
    base = c * _CB
    in_specs = [
        pl.BlockSpec((1, TD, EMB), lambda b: (b, 0, 0)),
        pl.BlockSpec((1, TQ, EMB), lambda b: (b + base, 0, 0)),
        pl.BlockSpec((1, 1, TD), lambda b: (b + base, 0, 0)),
        pl.BlockSpec((1, TQ, 1), lambda b: (b + base, 0, 0)),
        pl.BlockSpec((1, TQ, 1), lambda b: (b + base, 0, 0)),
        pl.BlockSpec((NODES, NBINS + 1), lambda b: (0, 0)),
        pl.BlockSpec((1, NODES), lambda b: (0, 0)),
        pl.BlockSpec((1, NODES), lambda b: (0, 0)),
        pl.BlockSpec(memory_space=pltpu.SMEM),
    ]
    return pl.pallas_call(
        _tc_body,
        grid=(_CB,),
        in_specs=in_specs,
        out_specs=pl.BlockSpec((1, 1, 1), lambda b: (b, 0, 0)),
        out_shape=jax.ShapeDtypeStruct((_CB, 1, 1), jnp.float32),
    )(d_emb_c, q_emb, sent3, qtok3, qidf3, W1, b1r, W2, consts)


def kernel(sentence, query_sentence, query_idf, emb, W1, b1, W2, b2, Wg, Wo, bo):
    sentence = sentence.astype(jnp.int32)
    query_sentence = query_sentence.astype(jnp.int32)

    didx = sentence.reshape(B * TD // _CHUNK, _CHUNK)
    qidx = query_sentence.reshape(_NW, _Q_ROWS_W)

    gather_d = _sc_gather_d()
    q_emb = _sc_gather_q()(emb, qidx).reshape(B, TQ, EMB)
    d_chunks = [
        gather_d(emb, didx[c * (_CB * TD // _CHUNK):(c + 1) * (_CB * TD // _CHUNK)]
                 ).reshape(_CB, TD, EMB)
        for c in range(_NCHUNK)
    ]

    sent3 = sentence.reshape(B, 1, TD)
    qtok3 = query_sentence.reshape(B, TQ, 1)
    qidf3 = query_idf.reshape(B, TQ, 1)
    b1r = b1.reshape(1, NODES)
    consts = jnp.stack([b2[0], Wg[0, 0], Wo[0, 0], bo[0]])

    scores = [
        _tc_chunk(c, d_chunks[c], q_emb, sent3, qtok3, qidf3, W1, b1r, W2,
                  consts)
        for c in range(_NCHUNK)
    ]
    return jnp.concatenate(scores, axis=0).reshape(B, 1)


# speedup vs baseline: 4.7012x; 1.1100x over previous
import functools

import numpy as np

import jax
import jax.numpy as jnp
from jax import lax
from jax.experimental import pallas as pl
from jax.experimental.pallas import tpu as pltpu
from jax.experimental.pallas import tpu_sc as plsc

B, TQ, TD = 128, 20, 2048
VOCAB, EMB = 100000, 128
NBINS, NODES = 30, 5

_UBS = [float(x) for x in np.linspace(-1.0, 1.0, NBINS + 1)[1:]]


_SC_CORES, _SC_SUBCORES = 2, 16
_NW = _SC_CORES * _SC_SUBCORES
_CHUNK = 128

_CB = 32
_NCHUNK = B // _CB
_DC_ROWS_W = (_CB * TD) // _NW
_DC_CHUNKS_W = _DC_ROWS_W // _CHUNK
_Q_ROWS_W = (B * TQ) // _NW


@functools.cache
def _sc_gather_d():
    @functools.partial(
        pl.kernel,
        out_type=jax.ShapeDtypeStruct((_CB * TD, EMB), jnp.float32),
        mesh=plsc.VectorSubcoreMesh(
            core_axis_name="c", subcore_axis_name="s", num_cores=_SC_CORES),
        scratch_types=[
            pltpu.VMEM((_DC_CHUNKS_W, _CHUNK), jnp.int32),
            pltpu.VMEM((_CHUNK, EMB), jnp.float32),
            pltpu.SemaphoreType.DMA,
        ],
    )
    def gather(table_hbm, didx_hbm, dout_hbm, didx_v, rows_v, sem):
        wid = lax.axis_index("s") * _SC_CORES + lax.axis_index("c")

        pltpu.sync_copy(
            didx_hbm.at[pl.ds(wid * _DC_CHUNKS_W, _DC_CHUNKS_W)], didx_v)

        def chunk_body(i, _):
            pltpu.async_copy(table_hbm.at[didx_v.at[i]], rows_v, sem).wait()
            pltpu.sync_copy(
                rows_v,
                dout_hbm.at[pl.ds((wid * _DC_CHUNKS_W + i) * _CHUNK, _CHUNK)])
            return 0

        lax.fori_loop(0, _DC_CHUNKS_W, chunk_body, 0)

    return gather


@functools.cache
def _sc_gather_q():
    @functools.partial(
        pl.kernel,
        out_type=jax.ShapeDtypeStruct((B * TQ, EMB), jnp.float32),
        mesh=plsc.VectorSubcoreMesh(
            core_axis_name="c", subcore_axis_name="s", num_cores=_SC_CORES),
        scratch_types=[
            pltpu.VMEM((_Q_ROWS_W,), jnp.int32),
            pltpu.VMEM((_Q_ROWS_W, EMB), jnp.float32),
            pltpu.SemaphoreType.DMA,
        ],
    )
    def gather(table_hbm, qidx_hbm, qout_hbm, qidx_v, qrows_v, sem):
        wid = lax.axis_index("s") * _SC_CORES + lax.axis_index("c")
        pltpu.sync_copy(qidx_hbm.at[wid], qidx_v)
        pltpu.async_copy(table_hbm.at[qidx_v], qrows_v, sem).wait()
        pltpu.sync_copy(qrows_v, qout_hbm.at[pl.ds(wid * _Q_ROWS_W, _Q_ROWS_W)])

    return gather


_BB = 2


def _score_one(de, qe, dtok, qtok, qidf, w1_ref, b1_ref, w2_ref, consts_ref):
    qn = qe / jnp.maximum(
        jnp.sqrt(jnp.sum(qe * qe, axis=1, keepdims=True)), 1e-8)

    ones_row = jnp.ones((1, EMB), jnp.float32)
    dnorm2 = lax.dot_general(ones_row, de * de, (((1,), (1,)), ((), ())),
                             preferred_element_type=jnp.float32)
    rd = 1.0 / jnp.maximum(jnp.sqrt(dnorm2), 1e-8)

    sim = lax.dot_general(qn, de, (((1,), (1,)), ((), ())),
                          preferred_element_type=jnp.float32)
    sim = sim * rd

    dmask0 = (dtok == 0).astype(jnp.float32)
    sim = sim + dmask0 * 1e7

    cum = [jnp.sum((sim < ub).astype(jnp.float32), axis=1, keepdims=True)
           for ub in _UBS]
    exact = jnp.sum(((sim > 0.999) & (sim < 1.001)).astype(jnp.float32),
                    axis=1, keepdims=True)
    cols = [cum[0]] + [cum[j] - cum[j - 1] for j in range(1, NBINS)] + [exact]
    hist = jnp.log(jnp.concatenate(cols, axis=1) + 1.0)

    h = jnp.tanh(
        lax.dot_general(hist, w1_ref[...], (((1,), (1,)), ((), ())),
                        preferred_element_type=jnp.float32) + b1_ref[...])
    b2s = consts_ref[0]
    wgs = consts_ref[1]
    wos = consts_ref[2]
    bos = consts_ref[3]

    ffw = jnp.tanh(
        jnp.sum(h * w2_ref[...], axis=1, keepdims=True) + b2s)

    qmask0 = (qtok == 0).astype(jnp.float32)
    logits = qidf * wgs + qmask0 * (-1e7)
    m = jnp.max(logits, axis=(0, 1), keepdims=True)
    e = jnp.exp(logits - m)
    w = e / jnp.sum(e, axis=(0, 1), keepdims=True)
    x = jnp.sum(w * ffw, axis=(0, 1), keepdims=True)
    return x * wos + bos


def _tc_body(d_emb_ref, q_emb_ref, d_tok_ref, q_tok_ref, q_idf_ref,
             w1_ref, b1_ref, w2_ref, consts_ref, out_ref):
    for bb in range(_BB):
        s = _score_one(d_emb_ref[bb], q_emb_ref[bb], d_tok_ref[bb],
                       q_tok_ref[bb], q_idf_ref[bb],
                       w1_ref, b1_ref, w2_ref, consts_ref)
        out_ref[bb] = s.reshape(1, 1)


def _tc_chunk(c, d_emb_c, q_emb, sent3, qtok3, qidf3, W1, b1r, W2, consts):
    base = c * (_CB // _BB)
    in_specs = [
        pl.BlockSpec((_BB, TD, EMB), lambda b: (b, 0, 0)),
        pl.BlockSpec((_BB, TQ, EMB), lambda b: (b + base, 0, 0)),
        pl.BlockSpec((_BB, 1, TD), lambda b: (b + base, 0, 0)),
        pl.BlockSpec((_BB, TQ, 1), lambda b: (b + base, 0, 0)),
        pl.BlockSpec((_BB, TQ, 1), lambda b: (b + base, 0, 0)),
        pl.BlockSpec((NODES, NBINS + 1), lambda b: (0, 0)),
        pl.BlockSpec((1, NODES), lambda b: (0, 0)),
        pl.BlockSpec((1, NODES), lambda b: (0, 0)),
        pl.BlockSpec(memory_space=pltpu.SMEM),
    ]
    return pl.pallas_call(
        _tc_body,
        grid=(_CB // _BB,),
        in_specs=in_specs,
        out_specs=pl.BlockSpec((_BB, 1, 1), lambda b: (b, 0, 0)),
        out_shape=jax.ShapeDtypeStruct((_CB, 1, 1), jnp.float32),
    )(d_emb_c, q_emb, sent3, qtok3, qidf3, W1, b1r, W2, consts)


def kernel(sentence, query_sentence, query_idf, emb, W1, b1, W2, b2, Wg, Wo, bo):
    sentence = sentence.astype(jnp.int32)
    query_sentence = query_sentence.astype(jnp.int32)

    didx = sentence.reshape(B * TD // _CHUNK, _CHUNK)
    qidx = query_sentence.reshape(_NW, _Q_ROWS_W)

    gather_d = _sc_gather_d()
    q_emb = _sc_gather_q()(emb, qidx).reshape(B, TQ, EMB)
    d_chunks = [
        gather_d(emb, didx[c * (_CB * TD // _CHUNK):(c + 1) * (_CB * TD // _CHUNK)]
                 ).reshape(_CB, TD, EMB)
        for c in range(_NCHUNK)
    ]

    sent3 = sentence.reshape(B, 1, TD)
    qtok3 = query_sentence.reshape(B, TQ, 1)
    qidf3 = query_idf.reshape(B, TQ, 1)
    b1r = b1.reshape(1, NODES)
    consts = jnp.stack([b2[0], Wg[0, 0], Wo[0, 0], bo[0]])

    scores = [
        _tc_chunk(c, d_chunks[c], q_emb, sent3, qtok3, qidf3, W1, b1r, W2,
                  consts)
        for c in range(_NCHUNK)
    ]
    return jnp.concatenate(scores, axis=0).reshape(B, 1)


# speedup vs baseline: 4.7951x; 1.0200x over previous
import functools

import numpy as np

import jax
import jax.numpy as jnp
from jax import lax
from jax.experimental import pallas as pl
from jax.experimental.pallas import tpu as pltpu
from jax.experimental.pallas import tpu_sc as plsc

B, TQ, TD = 128, 20, 2048
VOCAB, EMB = 100000, 128
NBINS, NODES = 30, 5

_UBS = [float(x) for x in np.linspace(-1.0, 1.0, NBINS + 1)[1:]]


_SC_CORES, _SC_SUBCORES = 2, 16
_NW = _SC_CORES * _SC_SUBCORES
_CHUNK = 128

_CB = 16
_NCHUNK = B // _CB
_DC_ROWS_W = (_CB * TD) // _NW
_DC_CHUNKS_W = _DC_ROWS_W // _CHUNK
_Q_ROWS_W = (B * TQ) // _NW


@functools.cache
def _sc_gather_d():
    @functools.partial(
        pl.kernel,
        out_type=jax.ShapeDtypeStruct((_CB * TD, EMB), jnp.float32),
        mesh=plsc.VectorSubcoreMesh(
            core_axis_name="c", subcore_axis_name="s", num_cores=_SC_CORES),
        scratch_types=[
            pltpu.VMEM((_DC_CHUNKS_W, _CHUNK), jnp.int32),
            pltpu.VMEM((_CHUNK, EMB), jnp.float32),
            pltpu.SemaphoreType.DMA,
        ],
    )
    def gather(table_hbm, didx_hbm, dout_hbm, didx_v, rows_v, sem):
        wid = lax.axis_index("s") * _SC_CORES + lax.axis_index("c")

        pltpu.sync_copy(
            didx_hbm.at[pl.ds(wid * _DC_CHUNKS_W, _DC_CHUNKS_W)], didx_v)

        def chunk_body(i, _):
            pltpu.async_copy(table_hbm.at[didx_v.at[i]], rows_v, sem).wait()
            pltpu.sync_copy(
                rows_v,
                dout_hbm.at[pl.ds((wid * _DC_CHUNKS_W + i) * _CHUNK, _CHUNK)])
            return 0

        lax.fori_loop(0, _DC_CHUNKS_W, chunk_body, 0)

    return gather


@functools.cache
def _sc_gather_q():
    @functools.partial(
        pl.kernel,
        out_type=jax.ShapeDtypeStruct((B * TQ, EMB), jnp.float32),
        mesh=plsc.VectorSubcoreMesh(
            core_axis_name="c", subcore_axis_name="s", num_cores=_SC_CORES),
        scratch_types=[
            pltpu.VMEM((_Q_ROWS_W,), jnp.int32),
            pltpu.VMEM((_Q_ROWS_W, EMB), jnp.float32),
            pltpu.SemaphoreType.DMA,
        ],
    )
    def gather(table_hbm, qidx_hbm, qout_hbm, qidx_v, qrows_v, sem):
        wid = lax.axis_index("s") * _SC_CORES + lax.axis_index("c")
        pltpu.sync_copy(qidx_hbm.at[wid], qidx_v)
        pltpu.async_copy(table_hbm.at[qidx_v], qrows_v, sem).wait()
        pltpu.sync_copy(qrows_v, qout_hbm.at[pl.ds(wid * _Q_ROWS_W, _Q_ROWS_W)])

    return gather


_BB = 4


def _score_one(de, qe, dtok, qtok, qidf, w1_ref, b1_ref, w2_ref, consts_ref):
    qn = qe / jnp.maximum(
        jnp.sqrt(jnp.sum(qe * qe, axis=1, keepdims=True)), 1e-8)

    ones_row = jnp.ones((1, EMB), jnp.float32)
    dnorm2 = lax.dot_general(ones_row, de * de, (((1,), (1,)), ((), ())),
                             preferred_element_type=jnp.float32)
    rd = 1.0 / jnp.maximum(jnp.sqrt(dnorm2), 1e-8)

    sim = lax.dot_general(qn, de, (((1,), (1,)), ((), ())),
                          preferred_element_type=jnp.float32)
    sim = sim * rd

    dmask0 = (dtok == 0).astype(jnp.float32)
    sim = sim + dmask0 * 1e7

    cum = [jnp.sum((sim < ub).astype(jnp.float32), axis=1, keepdims=True)
           for ub in _UBS]
    exact = jnp.sum(((sim > 0.999) & (sim < 1.001)).astype(jnp.float32),
                    axis=1, keepdims=True)
    cols = [cum[0]] + [cum[j] - cum[j - 1] for j in range(1, NBINS)] + [exact]
    hist = jnp.log(jnp.concatenate(cols, axis=1) + 1.0)

    h = jnp.tanh(
        lax.dot_general(hist, w1_ref[...], (((1,), (1,)), ((), ())),
                        preferred_element_type=jnp.float32) + b1_ref[...])
    b2s = consts_ref[0]
    wgs = consts_ref[1]
    wos = consts_ref[2]
    bos = consts_ref[3]

    ffw = jnp.tanh(
        jnp.sum(h * w2_ref[...], axis=1, keepdims=True) + b2s)

    qmask0 = (qtok == 0).astype(jnp.float32)
    logits = qidf * wgs + qmask0 * (-1e7)
    m = jnp.max(logits, axis=(0, 1), keepdims=True)
    e = jnp.exp(logits - m)
    w = e / jnp.sum(e, axis=(0, 1), keepdims=True)
    x = jnp.sum(w * ffw, axis=(0, 1), keepdims=True)
    return x * wos + bos


def _tc_body(d_emb_ref, q_emb_ref, d_tok_ref, q_tok_ref, q_idf_ref,
             w1_ref, b1_ref, w2_ref, consts_ref, out_ref):
    for bb in range(_BB):
        s = _score_one(d_emb_ref[bb], q_emb_ref[bb], d_tok_ref[bb],
                       q_tok_ref[bb], q_idf_ref[bb],
                       w1_ref, b1_ref, w2_ref, consts_ref)
        out_ref[bb] = s.reshape(1, 1)


def _tc_chunk(c, d_emb_c, q_emb, sent3, qtok3, qidf3, W1, b1r, W2, consts):
    base = c * (_CB // _BB)
    in_specs = [
        pl.BlockSpec((_BB, TD, EMB), lambda b: (b, 0, 0)),
        pl.BlockSpec((_BB, TQ, EMB), lambda b: (b + base, 0, 0)),
        pl.BlockSpec((_BB, 1, TD), lambda b: (b + base, 0, 0)),
        pl.BlockSpec((_BB, TQ, 1), lambda b: (b + base, 0, 0)),
        pl.BlockSpec((_BB, TQ, 1), lambda b: (b + base, 0, 0)),
        pl.BlockSpec((NODES, NBINS + 1), lambda b: (0, 0)),
        pl.BlockSpec((1, NODES), lambda b: (0, 0)),
        pl.BlockSpec((1, NODES), lambda b: (0, 0)),
        pl.BlockSpec(memory_space=pltpu.SMEM),
    ]
    return pl.pallas_call(
        _tc_body,
        grid=(_CB // _BB,),
        in_specs=in_specs,
        out_specs=pl.BlockSpec((_BB, 1, 1), lambda b: (b, 0, 0)),
        out_shape=jax.ShapeDtypeStruct((_CB, 1, 1), jnp.float32),
    )(d_emb_c, q_emb, sent3, qtok3, qidf3, W1, b1r, W2, consts)


def kernel(sentence, query_sentence, query_idf, emb, W1, b1, W2, b2, Wg, Wo, bo):
    sentence = sentence.astype(jnp.int32)
    query_sentence = query_sentence.astype(jnp.int32)

    didx = sentence.reshape(B * TD // _CHUNK, _CHUNK)
    qidx = query_sentence.reshape(_NW, _Q_ROWS_W)

    gather_d = _sc_gather_d()
    q_emb = _sc_gather_q()(emb, qidx).reshape(B, TQ, EMB)
    d_chunks = [
        gather_d(emb, didx[c * (_CB * TD // _CHUNK):(c + 1) * (_CB * TD // _CHUNK)]
                 ).reshape(_CB, TD, EMB)
        for c in range(_NCHUNK)
    ]

    sent3 = sentence.reshape(B, 1, TD)
    qtok3 = query_sentence.reshape(B, TQ, 1)
    qidf3 = query_idf.reshape(B, TQ, 1)
    b1r = b1.reshape(1, NODES)
    consts = jnp.stack([b2[0], Wg[0, 0], Wo[0, 0], bo[0]])

    scores = [
        _tc_chunk(c, d_chunks[c], q_emb, sent3, qtok3, qidf3, W1, b1r, W2,
                  consts)
        for c in range(_NCHUNK)
    ]
    return jnp.concatenate(scores, axis=0).reshape(B, 1)
